# Initial kernel scaffold; baseline (speedup 1.0000x reference)
import jax
import jax.numpy as jnp
from jax.experimental import pallas as pl


def kernel(x, edge_index, W1, W2, W3):
    raise NotImplementedError("write your pallas kernel here")


# speedup vs baseline: 7.5210x; 7.5210x over previous
import functools

import jax
import jax.numpy as jnp
from jax import lax
from jax.experimental import pallas as pl
from jax.experimental.pallas import tpu as pltpu
from jax.experimental.pallas import tpu_sc as plsc

N = 10000
E = 320000
D = 128
C = 6

DP = 16
NPAD = 10240
NCORES = 2
NSUB = 16
NW = NCORES * NSUB
EPAD = 327680
EPW = EPAD // NW
CHUNK = 128
NCHUNK = EPW // CHUNK
RPT = NPAD // NSUB


def _mm_body(x_ref, w1_ref, w2_ref, w3_ref, out_ref):
    hi = jax.lax.Precision.HIGHEST
    w12 = jnp.dot(w1_ref[...], w2_ref[...], precision=hi,
                  preferred_element_type=jnp.float32)
    w123 = jnp.dot(w12, w3_ref[...], precision=hi,
                   preferred_element_type=jnp.float32)
    out_ref[...] = jnp.dot(x_ref[...], w123, precision=hi,
                           preferred_element_type=jnp.float32)


def _add_body(p_ref, out_ref):
    out_ref[...] = p_ref[0] + p_ref[1]


_sc_mesh = plsc.VectorSubcoreMesh(core_axis_name="c", subcore_axis_name="s")


@functools.partial(
    pl.kernel,
    out_type=jax.ShapeDtypeStruct((NCORES * NPAD, DP), jnp.float32),
    mesh=_sc_mesh,
    scratch_types=[
        pltpu.VMEM((CHUNK,), jnp.int32),
        pltpu.VMEM((CHUNK,), jnp.int32),
        pltpu.VMEM((CHUNK, DP), jnp.float32),
        pltpu.VMEM_SHARED((NPAD, DP), jnp.float32),
        pltpu.SemaphoreType.DMA,
    ],
    compiler_params=pltpu.CompilerParams(use_tc_tiling_on_sc=False),
)
def _sc_pass(y_hbm, src_hbm, dst_hbm, zeros_hbm, out_hbm, sidx, didx, rows,
             acc, sem):
    c = lax.axis_index("c")
    s = lax.axis_index("s")
    wid = c * NSUB + s
    pltpu.sync_copy(zeros_hbm.at[pl.ds(s * RPT, RPT)],
                    acc.at[pl.ds(s * RPT, RPT)])
    plsc.subcore_barrier()
    base = wid * EPW

    def body(i, carry):
        off = base + i * CHUNK
        pltpu.sync_copy(src_hbm.at[pl.ds(off, CHUNK)], sidx)
        pltpu.sync_copy(dst_hbm.at[pl.ds(off, CHUNK)], didx)
        pltpu.async_copy(y_hbm.at[sidx], rows, sem).wait()
        pltpu.sync_copy(rows, acc.at[didx], add=True)
        return carry

    lax.fori_loop(0, NCHUNK, body, 0)
    plsc.subcore_barrier()
    pltpu.sync_copy(acc.at[pl.ds(s * RPT, RPT)],
                    out_hbm.at[pl.ds(c * NPAD + s * RPT, RPT)])


def kernel(x, edge_index, W1, W2, W3):
    src = edge_index[0]
    dst = edge_index[1]
    pad_e = EPAD - E
    src_p = jnp.concatenate([src, jnp.full((pad_e,), N, jnp.int32)])
    dst_p = jnp.concatenate([dst, jnp.full((pad_e,), N, jnp.int32)])
    x_p = jnp.pad(x, ((0, NPAD - N), (0, 0)))
    W3p = jnp.pad(W3, ((0, 0), (0, DP - C)))
    zeros = jnp.zeros((NPAD, DP), jnp.float32)

    y = pl.pallas_call(
        _mm_body,
        out_shape=jax.ShapeDtypeStruct((NPAD, DP), jnp.float32),
    )(x_p, W1, W2, W3p)

    for _ in range(3):
        p = _sc_pass(y, src_p, dst_p, zeros)
        p = p.reshape(NCORES, NPAD, DP)
        y = pl.pallas_call(
            _add_body,
            out_shape=jax.ShapeDtypeStruct((NPAD, DP), jnp.float32),
        )(p)

    return y[:N, :C]


# speedup vs baseline: 15.0343x; 1.9990x over previous
import functools

import jax
import jax.numpy as jnp
from jax import lax
from jax.experimental import pallas as pl
from jax.experimental.pallas import tpu as pltpu
from jax.experimental.pallas import tpu_sc as plsc

N = 10000
E = 320000
D = 128
C = 6

DP = 16
NPAD = 10240
NCORES = 2
NSUB = 16
NW = NCORES * NSUB
EPAD = 327680
EPW = EPAD // NW
CHUNK = 128
NCHUNK = EPW // CHUNK
RPT = NPAD // NSUB


def _mm_body(x_ref, w1_ref, w2_ref, w3_ref, out_ref):
    hi = jax.lax.Precision.HIGHEST
    w12 = jnp.dot(w1_ref[...], w2_ref[...], precision=hi,
                  preferred_element_type=jnp.float32)
    w123 = jnp.dot(w12, w3_ref[...], precision=hi,
                   preferred_element_type=jnp.float32)
    out_ref[...] = jnp.dot(x_ref[...], w123, precision=hi,
                           preferred_element_type=jnp.float32)


def _add_body(p_ref, out_ref):
    out_ref[...] = p_ref[0] + p_ref[1]


_sc_mesh = plsc.VectorSubcoreMesh(core_axis_name="c", subcore_axis_name="s")


NBUF = 4


@functools.partial(
    pl.kernel,
    out_type=jax.ShapeDtypeStruct((NCORES * NPAD, DP), jnp.float32),
    mesh=_sc_mesh,
    scratch_types=[
        pltpu.VMEM((NCHUNK, CHUNK), jnp.int32),
        pltpu.VMEM((NCHUNK, CHUNK), jnp.int32),
        [pltpu.VMEM((CHUNK, DP), jnp.float32) for _ in range(NBUF)],
        [pltpu.SemaphoreType.DMA for _ in range(NBUF)],
        pltpu.VMEM_SHARED((NPAD, DP), jnp.float32),
    ],
    compiler_params=pltpu.CompilerParams(use_tc_tiling_on_sc=False),
)
def _sc_pass(y_hbm, src_hbm, dst_hbm, zeros_hbm, out_hbm, sidx, didx, rows,
             gsem, acc):
    c = lax.axis_index("c")
    s = lax.axis_index("s")
    wid = c * NSUB + s
    pltpu.sync_copy(zeros_hbm.at[pl.ds(s * RPT, RPT)],
                    acc.at[pl.ds(s * RPT, RPT)])
    pltpu.sync_copy(src_hbm.at[wid], sidx)
    pltpu.sync_copy(dst_hbm.at[wid], didx)
    plsc.subcore_barrier()

    for b in range(NBUF):
        pltpu.async_copy(y_hbm.at[sidx.at[b]], rows[b], gsem[b])

    def group(g, carry):
        for b in range(NBUF):
            i = g * NBUF + b
            pltpu.make_async_copy(y_hbm.at[sidx.at[i]], rows[b],
                                  gsem[b]).wait()
            pltpu.sync_copy(rows[b], acc.at[didx.at[i]], add=True)

            @pl.when(i + NBUF < NCHUNK)
            def _():
                pltpu.async_copy(y_hbm.at[sidx.at[i + NBUF]], rows[b],
                                 gsem[b])
        return carry

    lax.fori_loop(0, NCHUNK // NBUF, group, 0)
    plsc.subcore_barrier()
    pltpu.sync_copy(acc.at[pl.ds(s * RPT, RPT)],
                    out_hbm.at[pl.ds(c * NPAD + s * RPT, RPT)])


def kernel(x, edge_index, W1, W2, W3):
    src = edge_index[0]
    dst = edge_index[1]
    pad_e = EPAD - E
    src_p = jnp.concatenate([src, jnp.full((pad_e,), N, jnp.int32)])
    dst_p = jnp.concatenate([dst, jnp.full((pad_e,), N, jnp.int32)])
    src_p = src_p.reshape(NW, NCHUNK, CHUNK)
    dst_p = dst_p.reshape(NW, NCHUNK, CHUNK)
    x_p = jnp.pad(x, ((0, NPAD - N), (0, 0)))
    W3p = jnp.pad(W3, ((0, 0), (0, DP - C)))
    zeros = jnp.zeros((NPAD, DP), jnp.float32)

    y = pl.pallas_call(
        _mm_body,
        out_shape=jax.ShapeDtypeStruct((NPAD, DP), jnp.float32),
    )(x_p, W1, W2, W3p)

    for _ in range(3):
        p = _sc_pass(y, src_p, dst_p, zeros)
        p = p.reshape(NCORES, NPAD, DP)
        y = pl.pallas_call(
            _add_body,
            out_shape=jax.ShapeDtypeStruct((NPAD, DP), jnp.float32),
        )(p)

    return y[:N, :C]


# speedup vs baseline: 15.0948x; 1.0040x over previous
import functools

import jax
import jax.numpy as jnp
from jax import lax
from jax.experimental import pallas as pl
from jax.experimental.pallas import tpu as pltpu
from jax.experimental.pallas import tpu_sc as plsc

N = 10000
E = 320000
D = 128
C = 6

DP = 16
NPAD = 10240
NCORES = 2
NSUB = 16
NW = NCORES * NSUB
EPAD = 327680
EPW = EPAD // NW
CHUNK = 128
NCHUNK = EPW // CHUNK
RPT = NPAD // NSUB


def _mm_body(x_ref, w1_ref, w2_ref, w3_ref, out_ref):
    hi = jax.lax.Precision.HIGHEST
    w12 = jnp.dot(w1_ref[...], w2_ref[...], precision=hi,
                  preferred_element_type=jnp.float32)
    w123 = jnp.dot(w12, w3_ref[...], precision=hi,
                   preferred_element_type=jnp.float32)
    out_ref[...] = jnp.dot(x_ref[...], w123, precision=hi,
                           preferred_element_type=jnp.float32)


def _add_body(p_ref, out_ref):
    out_ref[...] = p_ref[0] + p_ref[1]


_sc_mesh = plsc.VectorSubcoreMesh(core_axis_name="c", subcore_axis_name="s")


NBUF = 8


@functools.partial(
    pl.kernel,
    out_type=jax.ShapeDtypeStruct((NCORES * NPAD, DP), jnp.float32),
    mesh=_sc_mesh,
    scratch_types=[
        pltpu.VMEM((NCHUNK, CHUNK), jnp.int32),
        pltpu.VMEM((NCHUNK, CHUNK), jnp.int32),
        [pltpu.VMEM((CHUNK, DP), jnp.float32) for _ in range(NBUF)],
        [pltpu.SemaphoreType.DMA for _ in range(NBUF)],
        pltpu.VMEM_SHARED((NPAD, DP), jnp.float32),
    ],
    compiler_params=pltpu.CompilerParams(use_tc_tiling_on_sc=False),
)
def _sc_pass(y_hbm, src_hbm, dst_hbm, zeros_hbm, out_hbm, sidx, didx, rows,
             gsem, acc):
    c = lax.axis_index("c")
    s = lax.axis_index("s")
    wid = c * NSUB + s
    pltpu.sync_copy(zeros_hbm.at[pl.ds(s * RPT, RPT)],
                    acc.at[pl.ds(s * RPT, RPT)])
    pltpu.sync_copy(src_hbm.at[wid], sidx)
    pltpu.sync_copy(dst_hbm.at[wid], didx)
    plsc.subcore_barrier()

    for b in range(NBUF):
        pltpu.async_copy(y_hbm.at[sidx.at[b]], rows[b], gsem[b])

    def group(g, carry):
        for b in range(NBUF):
            i = g * NBUF + b
            pltpu.make_async_copy(y_hbm.at[sidx.at[i]], rows[b],
                                  gsem[b]).wait()
            pltpu.sync_copy(rows[b], acc.at[didx.at[i]], add=True)

            @pl.when(i + NBUF < NCHUNK)
            def _():
                pltpu.async_copy(y_hbm.at[sidx.at[i + NBUF]], rows[b],
                                 gsem[b])
        return carry

    lax.fori_loop(0, NCHUNK // NBUF, group, 0)
    plsc.subcore_barrier()
    pltpu.sync_copy(acc.at[pl.ds(s * RPT, RPT)],
                    out_hbm.at[pl.ds(c * NPAD + s * RPT, RPT)])


def kernel(x, edge_index, W1, W2, W3):
    src = edge_index[0]
    dst = edge_index[1]
    pad_e = EPAD - E
    src_p = jnp.concatenate([src, jnp.full((pad_e,), N, jnp.int32)])
    dst_p = jnp.concatenate([dst, jnp.full((pad_e,), N, jnp.int32)])
    src_p = src_p.reshape(NW, NCHUNK, CHUNK)
    dst_p = dst_p.reshape(NW, NCHUNK, CHUNK)
    x_p = jnp.pad(x, ((0, NPAD - N), (0, 0)))
    W3p = jnp.pad(W3, ((0, 0), (0, DP - C)))
    zeros = jnp.zeros((NPAD, DP), jnp.float32)

    y = pl.pallas_call(
        _mm_body,
        out_shape=jax.ShapeDtypeStruct((NPAD, DP), jnp.float32),
    )(x_p, W1, W2, W3p)

    for _ in range(3):
        p = _sc_pass(y, src_p, dst_p, zeros)
        p = p.reshape(NCORES, NPAD, DP)
        y = pl.pallas_call(
            _add_body,
            out_shape=jax.ShapeDtypeStruct((NPAD, DP), jnp.float32),
        )(p)

    return y[:N, :C]


# speedup vs baseline: 15.1266x; 1.0021x over previous
import functools

import jax
import jax.numpy as jnp
from jax import lax
from jax.experimental import pallas as pl
from jax.experimental.pallas import tpu as pltpu
from jax.experimental.pallas import tpu_sc as plsc

N = 10000
E = 320000
D = 128
C = 6

DP = 16
NPAD = 10240
NCORES = 2
NSUB = 16
NW = NCORES * NSUB
EPAD = 327680
EPW = EPAD // NW
CHUNK = 128
NCHUNK = EPW // CHUNK
RPT = NPAD // NSUB


def _mm_body(x_ref, w1_ref, w2_ref, w3_ref, out_ref):
    hi = jax.lax.Precision.HIGHEST
    w12 = jnp.dot(w1_ref[...], w2_ref[...], precision=hi,
                  preferred_element_type=jnp.float32)
    w123 = jnp.dot(w12, w3_ref[...], precision=hi,
                   preferred_element_type=jnp.float32)
    out_ref[...] = jnp.dot(x_ref[...], w123, precision=hi,
                           preferred_element_type=jnp.float32)


def _add_body(p_ref, out_ref):
    out_ref[...] = p_ref[0] + p_ref[1]


_sc_mesh = plsc.VectorSubcoreMesh(core_axis_name="c", subcore_axis_name="s")


NBUF = 8


@functools.partial(
    pl.kernel,
    out_type=jax.ShapeDtypeStruct((NCORES * NPAD, DP), jnp.float32),
    mesh=_sc_mesh,
    scratch_types=[
        pltpu.VMEM((NCHUNK, CHUNK), jnp.int32),
        pltpu.VMEM((NCHUNK, CHUNK), jnp.int32),
        [pltpu.VMEM((CHUNK, DP), jnp.float32) for _ in range(NBUF)],
        [pltpu.SemaphoreType.DMA for _ in range(NBUF)],
        [pltpu.SemaphoreType.DMA for _ in range(NBUF)],
        pltpu.VMEM_SHARED((NPAD, DP), jnp.float32),
    ],
    compiler_params=pltpu.CompilerParams(use_tc_tiling_on_sc=False),
)
def _sc_pass(y_hbm, src_hbm, dst_hbm, zeros_hbm, out_hbm, sidx, didx, rows,
             gsem, ssem, acc):
    c = lax.axis_index("c")
    s = lax.axis_index("s")
    wid = c * NSUB + s
    pltpu.sync_copy(zeros_hbm.at[pl.ds(s * RPT, RPT)],
                    acc.at[pl.ds(s * RPT, RPT)])
    pltpu.sync_copy(src_hbm.at[wid], sidx)
    pltpu.sync_copy(dst_hbm.at[wid], didx)
    plsc.subcore_barrier()

    for b in range(NBUF):
        pltpu.async_copy(y_hbm.at[sidx.at[b]], rows[b], gsem[b])

    def group(g, carry):
        for b in range(NBUF):
            i = g * NBUF + b
            bp = (b - 1) % NBUF
            pltpu.make_async_copy(y_hbm.at[sidx.at[i]], rows[b],
                                  gsem[b]).wait()
            pltpu.async_copy(rows[b], acc.at[didx.at[i]], ssem[b], add=True)

            @pl.when(i > 0)
            def _():
                pltpu.make_async_copy(rows[bp], acc.at[didx.at[i - 1]],
                                      ssem[bp]).wait()

                @pl.when(i - 1 + NBUF < NCHUNK)
                def _():
                    pltpu.async_copy(y_hbm.at[sidx.at[i - 1 + NBUF]],
                                     rows[bp], gsem[bp])
        return carry

    lax.fori_loop(0, NCHUNK // NBUF, group, 0)
    lastb = (NCHUNK - 1) % NBUF
    pltpu.make_async_copy(rows[lastb], acc.at[didx.at[NCHUNK - 1]],
                          ssem[lastb]).wait()
    plsc.subcore_barrier()
    pltpu.sync_copy(acc.at[pl.ds(s * RPT, RPT)],
                    out_hbm.at[pl.ds(c * NPAD + s * RPT, RPT)])


def kernel(x, edge_index, W1, W2, W3):
    src = edge_index[0]
    dst = edge_index[1]
    pad_e = EPAD - E
    src_p = jnp.concatenate([src, jnp.full((pad_e,), N, jnp.int32)])
    dst_p = jnp.concatenate([dst, jnp.full((pad_e,), N, jnp.int32)])
    src_p = src_p.reshape(NW, NCHUNK, CHUNK)
    dst_p = dst_p.reshape(NW, NCHUNK, CHUNK)
    x_p = jnp.pad(x, ((0, NPAD - N), (0, 0)))
    W3p = jnp.pad(W3, ((0, 0), (0, DP - C)))
    zeros = jnp.zeros((NPAD, DP), jnp.float32)

    y = pl.pallas_call(
        _mm_body,
        out_shape=jax.ShapeDtypeStruct((NPAD, DP), jnp.float32),
    )(x_p, W1, W2, W3p)

    for _ in range(3):
        p = _sc_pass(y, src_p, dst_p, zeros)
        p = p.reshape(NCORES, NPAD, DP)
        y = pl.pallas_call(
            _add_body,
            out_shape=jax.ShapeDtypeStruct((NPAD, DP), jnp.float32),
        )(p)

    return y[:N, :C]


# speedup vs baseline: 19.1357x; 1.2650x over previous
import functools

import jax
import jax.numpy as jnp
from jax import lax
from jax.experimental import pallas as pl
from jax.experimental.pallas import tpu as pltpu
from jax.experimental.pallas import tpu_sc as plsc

N = 10000
E = 320000
D = 128
C = 6

DP = 8
NPAD = 10240
NCORES = 2
NSUB = 16
NW = NCORES * NSUB
EPAD = 327680
EPW = EPAD // NW
CHUNK = 128
NCHUNK = EPW // CHUNK
RPT = NPAD // NSUB


def _mm_body(x_ref, w1_ref, w2_ref, w3_ref, out_ref):
    hi = jax.lax.Precision.HIGHEST
    w12 = jnp.dot(w1_ref[...], w2_ref[...], precision=hi,
                  preferred_element_type=jnp.float32)
    w123 = jnp.dot(w12, w3_ref[...], precision=hi,
                   preferred_element_type=jnp.float32)
    out_ref[...] = jnp.dot(x_ref[...], w123, precision=hi,
                           preferred_element_type=jnp.float32)


def _add_body(p_ref, out_ref):
    out_ref[...] = p_ref[0] + p_ref[1]


_sc_mesh = plsc.VectorSubcoreMesh(core_axis_name="c", subcore_axis_name="s")


NBUF = 8


@functools.partial(
    pl.kernel,
    out_type=jax.ShapeDtypeStruct((NCORES * NPAD, DP), jnp.float32),
    mesh=_sc_mesh,
    scratch_types=[
        pltpu.VMEM((NCHUNK, CHUNK), jnp.int32),
        pltpu.VMEM((NCHUNK, CHUNK), jnp.int32),
        [pltpu.VMEM((CHUNK, DP), jnp.float32) for _ in range(NBUF)],
        [pltpu.SemaphoreType.DMA for _ in range(NBUF)],
        [pltpu.SemaphoreType.DMA for _ in range(NBUF)],
        pltpu.VMEM_SHARED((NPAD, DP), jnp.float32),
    ],
    compiler_params=pltpu.CompilerParams(use_tc_tiling_on_sc=False),
)
def _sc_pass(y_hbm, src_hbm, dst_hbm, zeros_hbm, out_hbm, sidx, didx, rows,
             gsem, ssem, acc):
    c = lax.axis_index("c")
    s = lax.axis_index("s")
    wid = c * NSUB + s
    pltpu.sync_copy(zeros_hbm.at[pl.ds(s * RPT, RPT)],
                    acc.at[pl.ds(s * RPT, RPT)])
    pltpu.sync_copy(src_hbm.at[wid], sidx)
    pltpu.sync_copy(dst_hbm.at[wid], didx)
    plsc.subcore_barrier()

    for b in range(NBUF):
        pltpu.async_copy(y_hbm.at[sidx.at[b]], rows[b], gsem[b])

    def group(g, carry):
        for b in range(NBUF):
            i = g * NBUF + b
            bp = (b - 1) % NBUF
            pltpu.make_async_copy(y_hbm.at[sidx.at[i]], rows[b],
                                  gsem[b]).wait()
            pltpu.async_copy(rows[b], acc.at[didx.at[i]], ssem[b], add=True)

            @pl.when(i > 0)
            def _():
                pltpu.make_async_copy(rows[bp], acc.at[didx.at[i - 1]],
                                      ssem[bp]).wait()

                @pl.when(i - 1 + NBUF < NCHUNK)
                def _():
                    pltpu.async_copy(y_hbm.at[sidx.at[i - 1 + NBUF]],
                                     rows[bp], gsem[bp])
        return carry

    lax.fori_loop(0, NCHUNK // NBUF, group, 0)
    lastb = (NCHUNK - 1) % NBUF
    pltpu.make_async_copy(rows[lastb], acc.at[didx.at[NCHUNK - 1]],
                          ssem[lastb]).wait()
    plsc.subcore_barrier()
    pltpu.sync_copy(acc.at[pl.ds(s * RPT, RPT)],
                    out_hbm.at[pl.ds(c * NPAD + s * RPT, RPT)])


def kernel(x, edge_index, W1, W2, W3):
    src = edge_index[0]
    dst = edge_index[1]
    pad_e = EPAD - E
    src_p = jnp.concatenate([src, jnp.full((pad_e,), N, jnp.int32)])
    dst_p = jnp.concatenate([dst, jnp.full((pad_e,), N, jnp.int32)])
    src_p = src_p.reshape(NW, NCHUNK, CHUNK)
    dst_p = dst_p.reshape(NW, NCHUNK, CHUNK)
    x_p = jnp.pad(x, ((0, NPAD - N), (0, 0)))
    W3p = jnp.pad(W3, ((0, 0), (0, DP - C)))
    zeros = jnp.zeros((NPAD, DP), jnp.float32)

    y = pl.pallas_call(
        _mm_body,
        out_shape=jax.ShapeDtypeStruct((NPAD, DP), jnp.float32),
    )(x_p, W1, W2, W3p)

    for _ in range(3):
        p = _sc_pass(y, src_p, dst_p, zeros)
        p = p.reshape(NCORES, NPAD, DP)
        y = pl.pallas_call(
            _add_body,
            out_shape=jax.ShapeDtypeStruct((NPAD, DP), jnp.float32),
        )(p)

    return y[:N, :C]


# speedup vs baseline: 33.5783x; 1.7547x over previous
import functools

import jax
import jax.numpy as jnp
from jax import lax
from jax.experimental import pallas as pl
from jax.experimental.pallas import tpu as pltpu
from jax.experimental.pallas import tpu_sc as plsc

N = 10000
E = 320000
D = 128
C = 6

DP = 8
NPAD = 10240
NCORES = 2
NSUB = 16
NW = NCORES * NSUB
EPAD = 327680
EPW = EPAD // NW
CHUNK = 128
NCHUNK = EPW // CHUNK
RPT = NPAD // NSUB


def _mm_body(x_ref, w1_ref, w2_ref, w3_ref, out_ref):
    hi = jax.lax.Precision.HIGHEST
    w12 = jnp.dot(w1_ref[...], w2_ref[...], precision=hi,
                  preferred_element_type=jnp.float32)
    w123 = jnp.dot(w12, w3_ref[...], precision=hi,
                   preferred_element_type=jnp.float32)
    y0 = jnp.dot(x_ref[...], w123, precision=hi,
                 preferred_element_type=jnp.float32)
    out_ref[...] = jnp.concatenate(
        [y0, jnp.zeros_like(y0)], axis=0)


def _add_body(p_ref, out_ref):
    out_ref[...] = p_ref[0] + p_ref[1]


_sc_mesh = plsc.VectorSubcoreMesh(core_axis_name="c", subcore_axis_name="s")


NBUF = 8


NIOTA = RPT // CHUNK


@functools.partial(
    pl.kernel,
    out_type=jax.ShapeDtypeStruct((NCORES * NPAD, DP), jnp.float32),
    mesh=_sc_mesh,
    scratch_types=[
        pltpu.VMEM((NCHUNK, CHUNK), jnp.int32),
        pltpu.VMEM((NCHUNK, CHUNK), jnp.int32),
        [pltpu.VMEM((CHUNK, DP), jnp.float32) for _ in range(NBUF)],
        [pltpu.SemaphoreType.DMA for _ in range(NBUF)],
        [pltpu.SemaphoreType.DMA for _ in range(NBUF)],
        pltpu.VMEM((NIOTA, CHUNK), jnp.int32),
        pltpu.VMEM((RPT, DP), jnp.float32),
        pltpu.VMEM_SHARED((NPAD, DP), jnp.float32),
        pltpu.VMEM_SHARED((NPAD, DP), jnp.float32),
    ],
    compiler_params=pltpu.CompilerParams(use_tc_tiling_on_sc=False),
)
def _sc_pass(p_hbm, src_hbm, dst_hbm, zeros_hbm, iota_hbm, out_hbm,
             sidx, didx, rows, gsem, ssem, iidx, pbuf, yspm, acc):
    c = lax.axis_index("c")
    s = lax.axis_index("s")
    wid = c * NSUB + s
    pltpu.sync_copy(p_hbm.at[pl.ds(s * RPT, RPT)],
                    yspm.at[pl.ds(s * RPT, RPT)])
    pltpu.sync_copy(zeros_hbm.at[pl.ds(s * RPT, RPT)],
                    acc.at[pl.ds(s * RPT, RPT)])
    pltpu.sync_copy(p_hbm.at[pl.ds(NPAD + s * RPT, RPT)], pbuf)
    pltpu.sync_copy(iota_hbm.at[s], iidx)
    pltpu.sync_copy(src_hbm.at[wid], sidx)
    pltpu.sync_copy(dst_hbm.at[wid], didx)
    for j in range(NIOTA):
        pltpu.sync_copy(pbuf.at[pl.ds(j * CHUNK, CHUNK)],
                        yspm.at[iidx.at[j]], add=True)
    plsc.subcore_barrier()

    for b in range(NBUF):
        pltpu.async_copy(yspm.at[sidx.at[b]], rows[b], gsem[b])

    def group(g, carry):
        for b in range(NBUF):
            i = g * NBUF + b
            bp = (b - 1) % NBUF
            pltpu.make_async_copy(yspm.at[sidx.at[i]], rows[b],
                                  gsem[b]).wait()
            pltpu.async_copy(rows[b], acc.at[didx.at[i]], ssem[b], add=True)

            @pl.when(i > 0)
            def _():
                pltpu.make_async_copy(rows[bp], acc.at[didx.at[i - 1]],
                                      ssem[bp]).wait()

                @pl.when(i - 1 + NBUF < NCHUNK)
                def _():
                    pltpu.async_copy(yspm.at[sidx.at[i - 1 + NBUF]],
                                     rows[bp], gsem[bp])
        return carry

    lax.fori_loop(0, NCHUNK // NBUF, group, 0)
    lastb = (NCHUNK - 1) % NBUF
    pltpu.make_async_copy(rows[lastb], acc.at[didx.at[NCHUNK - 1]],
                          ssem[lastb]).wait()
    plsc.subcore_barrier()
    pltpu.sync_copy(acc.at[pl.ds(s * RPT, RPT)],
                    out_hbm.at[pl.ds(c * NPAD + s * RPT, RPT)])


def kernel(x, edge_index, W1, W2, W3):
    src = edge_index[0]
    dst = edge_index[1]
    pad_e = EPAD - E
    src_p = jnp.concatenate([src, jnp.full((pad_e,), N, jnp.int32)])
    dst_p = jnp.concatenate([dst, jnp.full((pad_e,), N, jnp.int32)])
    src_p = src_p.reshape(NW, NCHUNK, CHUNK)
    dst_p = dst_p.reshape(NW, NCHUNK, CHUNK)
    x_p = jnp.pad(x, ((0, NPAD - N), (0, 0)))
    W3p = jnp.pad(W3, ((0, 0), (0, DP - C)))
    zeros = jnp.zeros((NPAD, DP), jnp.float32)
    iota = jnp.arange(NPAD, dtype=jnp.int32).reshape(NSUB, NIOTA, CHUNK)

    p = pl.pallas_call(
        _mm_body,
        out_shape=jax.ShapeDtypeStruct((NCORES * NPAD, DP), jnp.float32),
    )(x_p, W1, W2, W3p)

    for _ in range(3):
        p = _sc_pass(p, src_p, dst_p, zeros, iota)

    y = pl.pallas_call(
        _add_body,
        out_shape=jax.ShapeDtypeStruct((NPAD, DP), jnp.float32),
    )(p.reshape(NCORES, NPAD, DP))

    return y[:N, :C]


# speedup vs baseline: 34.7178x; 1.0339x over previous
import functools

import jax
import jax.numpy as jnp
from jax import lax
from jax.experimental import pallas as pl
from jax.experimental.pallas import tpu as pltpu
from jax.experimental.pallas import tpu_sc as plsc

N = 10000
E = 320000
D = 128
C = 6

DP = 8
NPAD = 10240
NCORES = 2
NSUB = 16
NW = NCORES * NSUB
EPAD = 327680
EPW = EPAD // NW
CHUNK = 128
NCHUNK = EPW // CHUNK
RPT = NPAD // NSUB


def _mm_body(x_ref, w1_ref, w2_ref, w3_ref, out_ref):
    hi = jax.lax.Precision.HIGHEST
    w12 = jnp.dot(w1_ref[...], w2_ref[...], precision=hi,
                  preferred_element_type=jnp.float32)
    w123 = jnp.dot(w12, w3_ref[...], precision=hi,
                   preferred_element_type=jnp.float32)
    y0 = jnp.dot(x_ref[...], w123, precision=hi,
                 preferred_element_type=jnp.float32)
    out_ref[...] = jnp.pad(y0, ((0, 2 * NPAD - N), (0, 0)))


def _add_body(p_ref, out_ref):
    out_ref[...] = p_ref[0, :N] + p_ref[1, :N]


_sc_mesh = plsc.VectorSubcoreMesh(core_axis_name="c", subcore_axis_name="s")


NBUF = 8


NIOTA = RPT // CHUNK


@functools.partial(
    pl.kernel,
    out_type=jax.ShapeDtypeStruct((NCORES * NPAD, DP), jnp.float32),
    mesh=_sc_mesh,
    scratch_types=[
        pltpu.VMEM((NCHUNK, CHUNK), jnp.int32),
        pltpu.VMEM((NCHUNK, CHUNK), jnp.int32),
        [pltpu.VMEM((CHUNK, DP), jnp.float32) for _ in range(NBUF)],
        [pltpu.SemaphoreType.DMA for _ in range(NBUF)],
        [pltpu.SemaphoreType.DMA for _ in range(NBUF)],
        pltpu.VMEM((NIOTA, CHUNK), jnp.int32),
        pltpu.VMEM((RPT, DP), jnp.float32),
        pltpu.VMEM_SHARED((NPAD, DP), jnp.float32),
        pltpu.VMEM_SHARED((NPAD, DP), jnp.float32),
    ],
    compiler_params=pltpu.CompilerParams(use_tc_tiling_on_sc=False),
)
def _sc_pass(p_hbm, src_hbm, dst_hbm, zeros_hbm, iota_hbm, out_hbm,
             sidx, didx, rows, gsem, ssem, iidx, pbuf, yspm, acc):
    c = lax.axis_index("c")
    s = lax.axis_index("s")
    wid = c * NSUB + s
    pltpu.sync_copy(p_hbm.at[pl.ds(s * RPT, RPT)],
                    yspm.at[pl.ds(s * RPT, RPT)])
    pltpu.sync_copy(zeros_hbm.at[pl.ds(s * RPT, RPT)],
                    acc.at[pl.ds(s * RPT, RPT)])
    pltpu.sync_copy(p_hbm.at[pl.ds(NPAD + s * RPT, RPT)], pbuf)
    pltpu.sync_copy(iota_hbm.at[s], iidx)
    pltpu.sync_copy(src_hbm.at[wid], sidx)
    pltpu.sync_copy(dst_hbm.at[wid], didx)
    for j in range(NIOTA):
        pltpu.sync_copy(pbuf.at[pl.ds(j * CHUNK, CHUNK)],
                        yspm.at[iidx.at[j]], add=True)
    plsc.subcore_barrier()

    for b in range(NBUF):
        pltpu.async_copy(yspm.at[sidx.at[b]], rows[b], gsem[b])

    def group(g, carry):
        for b in range(NBUF):
            i = g * NBUF + b
            bp = (b - 1) % NBUF
            pltpu.make_async_copy(yspm.at[sidx.at[i]], rows[b],
                                  gsem[b]).wait()
            pltpu.async_copy(rows[b], acc.at[didx.at[i]], ssem[b], add=True)

            @pl.when(i > 0)
            def _():
                pltpu.make_async_copy(rows[bp], acc.at[didx.at[i - 1]],
                                      ssem[bp]).wait()

                @pl.when(i - 1 + NBUF < NCHUNK)
                def _():
                    pltpu.async_copy(yspm.at[sidx.at[i - 1 + NBUF]],
                                     rows[bp], gsem[bp])
        return carry

    lax.fori_loop(0, NCHUNK // NBUF, group, 0)
    lastb = (NCHUNK - 1) % NBUF
    pltpu.make_async_copy(rows[lastb], acc.at[didx.at[NCHUNK - 1]],
                          ssem[lastb]).wait()
    plsc.subcore_barrier()
    pltpu.sync_copy(acc.at[pl.ds(s * RPT, RPT)],
                    out_hbm.at[pl.ds(c * NPAD + s * RPT, RPT)])


def kernel(x, edge_index, W1, W2, W3):
    src = edge_index[0]
    dst = edge_index[1]
    pad_e = EPAD - E
    src_p = jnp.concatenate([src, jnp.full((pad_e,), N, jnp.int32)])
    dst_p = jnp.concatenate([dst, jnp.full((pad_e,), N, jnp.int32)])
    src_p = src_p.reshape(NW, NCHUNK, CHUNK)
    dst_p = dst_p.reshape(NW, NCHUNK, CHUNK)
    W3p = jnp.pad(W3, ((0, 0), (0, DP - C)))
    zeros = jnp.zeros((NPAD, DP), jnp.float32)
    iota = jnp.arange(NPAD, dtype=jnp.int32).reshape(NSUB, NIOTA, CHUNK)

    p = pl.pallas_call(
        _mm_body,
        out_shape=jax.ShapeDtypeStruct((NCORES * NPAD, DP), jnp.float32),
    )(x, W1, W2, W3p)

    for _ in range(3):
        p = _sc_pass(p, src_p, dst_p, zeros, iota)

    y = pl.pallas_call(
        _add_body,
        out_shape=jax.ShapeDtypeStruct((N, DP), jnp.float32),
    )(p.reshape(NCORES, NPAD, DP))

    return y[:, :C]
